# baseline (speedup 1.0000x reference)
import functools

import jax
import jax.numpy as jnp
from jax import lax
from jax.experimental import pallas as pl
from jax.experimental.pallas import tpu as pltpu
from jax.experimental.pallas import tpu_sc as plsc

_NC = 2
_NS = 16
_NW = _NC * _NS
_CHUNK = 400


def _proj_body(nodes_ref, wr_ref, ws_ref, glob_ref, wg_ref, b_ref,
               pr_ref, ps_ref, c_ref):
    n = nodes_ref[...]
    hi = jax.lax.Precision.HIGHEST
    pr_ref[...] = jnp.dot(n, wr_ref[...], precision=hi,
                          preferred_element_type=jnp.float32)
    ps_ref[...] = jnp.dot(n, ws_ref[...], precision=hi,
                          preferred_element_type=jnp.float32)
    c_ref[...] = jnp.dot(glob_ref[...], wg_ref[...], precision=hi,
                         preferred_element_type=jnp.float32) + b_ref[...]


def _sc_body(epw, nchunk, pack, d_out, pr_hbm, ps_hbm, recv_hbm, send_hbm,
             g_hbm, ridx, sidx, rrows0, srows0, rrows1, srows1,
             sem_r0, sem_s0, sem_st0, sem_r1, sem_s1, sem_st1):
    wid = lax.axis_index("s") * _NC + lax.axis_index("c")
    base = wid * epw
    bufs = ((rrows0, srows0, sem_r0, sem_s0, sem_st0),
            (rrows1, srows1, sem_r1, sem_s1, sem_st1))

    pltpu.sync_copy(recv_hbm.at[pl.ds(base, epw)], ridx)
    pltpu.sync_copy(send_hbm.at[pl.ds(base, epw)], sidx)

    def issue(ci, b):
        rr, sr, semr, sems, _ = bufs[b]
        sl = pl.ds(ci * _CHUNK, _CHUNK)
        pltpu.async_copy(pr_hbm.at[ridx.at[sl]], rr, semr)
        pltpu.async_copy(ps_hbm.at[sidx.at[sl]], sr, sems)

    def finish(ci, b):
        rr, sr, semr, sems, semst = bufs[b]
        sl = pl.ds(0, _CHUNK)
        pltpu.make_async_copy(pr_hbm.at[ridx.at[sl]], rr, semr).wait()
        pltpu.make_async_copy(ps_hbm.at[sidx.at[sl]], sr, sems).wait()

        def row_body(r, carry2):
            e = r * 4
            for k in range(4):
                rr[e + k, :] = rr[e + k, :] + sr[e + k, :]
            return carry2

        lax.fori_loop(0, _CHUNK // 4, row_body, 0)
        c = (base + ci * _CHUNK) // _CHUNK
        band = c % pack
        r0 = (c // pack) * _CHUNK
        pltpu.async_copy(
            rr, g_hbm.at[pl.ds(r0, _CHUNK), pl.ds(band * d_out, d_out)],
            semst)

    def drain_store(b):
        rr, _, _, _, semst = bufs[b]
        pltpu.make_async_copy(
            rr, g_hbm.at[pl.ds(0, _CHUNK), pl.ds(0, d_out)], semst).wait()

    issue(0, 0)
    issue(1, 1)

    def pair_body(j, carry):
        for h in range(2):
            ci = j * 2 + h

            finish(ci, h)

            @pl.when(ci + 2 < nchunk)
            def _(h=h, ci=ci):
                drain_store(h)
                issue(ci + 2, h)

        return carry

    lax.fori_loop(0, (nchunk - 1) // 2, pair_body, 0)
    finish(nchunk - 1, 0)
    drain_store(0)
    drain_store(1)


def _combine_body(pack, et_ref, g_ref, wet_ref, ct_ref, eye_ref, ot_ref):
    hi = jax.lax.Precision.HIGHEST
    edge_term = jnp.dot(wet_ref[...], et_ref[...], precision=hi,
                        preferred_element_type=jnp.float32)
    d_out = 128 // pack
    ct = ct_ref[...]
    for k in range(pack):
        g_t = jax.lax.dot_general(
            eye_ref[...], g_ref[:, k * d_out:(k + 1) * d_out],
            (((1,), (1,)), ((), ())),
            preferred_element_type=jnp.float32)
        ot_ref[:, k * _CHUNK:(k + 1) * _CHUNK] = (
            edge_term[:, k * _CHUNK:(k + 1) * _CHUNK] + g_t + ct)


def kernel(edges, nodes, globals_, receivers, senders, W, b):
    E, d_edge = edges.shape
    N, d_node = nodes.shape
    d_out = W.shape[-1]
    f32 = jnp.float32
    pack = 128 // d_out

    we = W[:d_edge]
    wr = W[d_edge:d_edge + d_node]
    ws = W[d_edge + d_node:d_edge + 2 * d_node]
    wg = W[d_edge + 2 * d_node:]
    b2 = b.reshape(1, d_out)

    recv32 = receivers.astype(jnp.int32)
    send32 = senders.astype(jnp.int32)

    pr, ps, c = pl.pallas_call(
        _proj_body,
        out_shape=[jax.ShapeDtypeStruct((N, d_out), f32),
                   jax.ShapeDtypeStruct((N, d_out), f32),
                   jax.ShapeDtypeStruct((1, d_out), f32)],
    )(nodes, wr, ws, globals_, wg, b2)

    epw = E // _NW
    nchunk = epw // _CHUNK
    grows = E // pack
    mesh = plsc.VectorSubcoreMesh(core_axis_name="c", subcore_axis_name="s")
    sc = pl.kernel(
        functools.partial(_sc_body, epw, nchunk, pack, d_out),
        mesh=mesh,
        compiler_params=pltpu.CompilerParams(use_tc_tiling_on_sc=False),
        out_type=jax.ShapeDtypeStruct((grows, 128), f32),
        scratch_types=[
            pltpu.VMEM((epw,), jnp.int32),
            pltpu.VMEM((epw,), jnp.int32),
            pltpu.VMEM((_CHUNK, d_out), f32),
            pltpu.VMEM((_CHUNK, d_out), f32),
            pltpu.VMEM((_CHUNK, d_out), f32),
            pltpu.VMEM((_CHUNK, d_out), f32),
            pltpu.SemaphoreType.DMA,
            pltpu.SemaphoreType.DMA,
            pltpu.SemaphoreType.DMA,
            pltpu.SemaphoreType.DMA,
            pltpu.SemaphoreType.DMA,
            pltpu.SemaphoreType.DMA,
        ],
    )
    g = sc(pr, ps, recv32, send32)

    et = edges.T
    wet = we.T
    ct = c.T
    eye16 = jnp.eye(d_out, dtype=f32)
    eb = pack * _CHUNK
    out_t = pl.pallas_call(
        functools.partial(_combine_body, pack),
        grid=(E // eb,),
        in_specs=[
            pl.BlockSpec((d_out, eb), lambda i: (0, i)),
            pl.BlockSpec((_CHUNK, 128), lambda i: (i, 0)),
            pl.BlockSpec((d_out, d_out), lambda i: (0, 0)),
            pl.BlockSpec((d_out, 1), lambda i: (0, 0)),
            pl.BlockSpec((d_out, d_out), lambda i: (0, 0)),
        ],
        out_specs=pl.BlockSpec((d_out, eb), lambda i: (0, i)),
        out_shape=jax.ShapeDtypeStruct((d_out, E), f32),
    )(et, g, wet, ct, eye16)
    return out_t.T


# speedup vs baseline: 1.3046x; 1.3046x over previous
import functools

import jax
import jax.numpy as jnp
from jax import lax
from jax.experimental import pallas as pl
from jax.experimental.pallas import tpu as pltpu
from jax.experimental.pallas import tpu_sc as plsc

_NC = 2
_NS = 16
_NW = _NC * _NS
_CHUNK = 400
_BQ = 2000


def _proj_body(nodes_ref, wr_ref, ws_ref, glob_ref, wg_ref, b_ref,
               pr_ref, ps_ref, c_ref):
    n = nodes_ref[...]
    hi = jax.lax.Precision.HIGHEST
    pr_ref[...] = jnp.dot(n, wr_ref[...], precision=hi,
                          preferred_element_type=jnp.float32)
    ps_ref[...] = jnp.dot(n, ws_ref[...], precision=hi,
                          preferred_element_type=jnp.float32)
    c_ref[...] = jnp.dot(glob_ref[...], wg_ref[...], precision=hi,
                         preferred_element_type=jnp.float32) + b_ref[...]


def _sc_body(epw, nchunk, pack, d_out, pr_hbm, ps_hbm, recv_hbm, send_hbm,
             g_hbm, ridx, sidx, rrows0, srows0, rrows1, srows1,
             sem_r0, sem_s0, sem_st0, sem_r1, sem_s1, sem_st1):
    wid = lax.axis_index("s") * _NC + lax.axis_index("c")
    base = wid * epw
    bufs = ((rrows0, srows0, sem_r0, sem_s0, sem_st0),
            (rrows1, srows1, sem_r1, sem_s1, sem_st1))

    pltpu.sync_copy(recv_hbm.at[pl.ds(base, epw)], ridx)
    pltpu.sync_copy(send_hbm.at[pl.ds(base, epw)], sidx)

    def issue(ci, b):
        rr, sr, semr, sems, _ = bufs[b]
        sl = pl.ds(ci * _CHUNK, _CHUNK)
        pltpu.async_copy(pr_hbm.at[ridx.at[sl]], rr, semr)
        pltpu.async_copy(ps_hbm.at[sidx.at[sl]], sr, sems)

    def finish(ci, b):
        rr, sr, semr, sems, semst = bufs[b]
        sl = pl.ds(0, _CHUNK)
        pltpu.make_async_copy(pr_hbm.at[ridx.at[sl]], rr, semr).wait()
        pltpu.make_async_copy(ps_hbm.at[sidx.at[sl]], sr, sems).wait()

        def row_body(r, carry2):
            e = r * 4
            for k in range(4):
                rr[e + k, :] = rr[e + k, :] + sr[e + k, :]
            return carry2

        lax.fori_loop(0, _CHUNK // 4, row_body, 0)
        off = base + ci * _CHUNK
        band = (off // _BQ) % pack
        r0 = (off // (_BQ * pack)) * _BQ + (off % _BQ)
        pltpu.async_copy(
            rr, g_hbm.at[pl.ds(r0, _CHUNK), pl.ds(band * d_out, d_out)],
            semst)

    def drain_store(b):
        rr, _, _, _, semst = bufs[b]
        pltpu.make_async_copy(
            rr, g_hbm.at[pl.ds(0, _CHUNK), pl.ds(0, d_out)], semst).wait()

    issue(0, 0)
    issue(1, 1)

    def pair_body(j, carry):
        for h in range(2):
            ci = j * 2 + h

            finish(ci, h)

            @pl.when(ci + 2 < nchunk)
            def _(h=h, ci=ci):
                drain_store(h)
                issue(ci + 2, h)

        return carry

    lax.fori_loop(0, (nchunk - 1) // 2, pair_body, 0)
    finish(nchunk - 1, 0)
    drain_store(0)
    drain_store(1)


def _combine_body(pack, et_ref, g_ref, wet_ref, ct_ref, eye_ref, ot_ref):
    hi = jax.lax.Precision.HIGHEST
    edge_term = jnp.dot(wet_ref[...], et_ref[...], precision=hi,
                        preferred_element_type=jnp.float32)
    d_out = 128 // pack
    ct = ct_ref[...]
    for k in range(pack):
        g_t = jax.lax.dot_general(
            eye_ref[...], g_ref[:, k * d_out:(k + 1) * d_out],
            (((1,), (1,)), ((), ())),
            preferred_element_type=jnp.float32)
        ot_ref[:, k * _BQ:(k + 1) * _BQ] = (
            edge_term[:, k * _BQ:(k + 1) * _BQ] + g_t + ct)


def kernel(edges, nodes, globals_, receivers, senders, W, b):
    E, d_edge = edges.shape
    N, d_node = nodes.shape
    d_out = W.shape[-1]
    f32 = jnp.float32
    pack = 128 // d_out

    we = W[:d_edge]
    wr = W[d_edge:d_edge + d_node]
    ws = W[d_edge + d_node:d_edge + 2 * d_node]
    wg = W[d_edge + 2 * d_node:]
    b2 = b.reshape(1, d_out)

    recv32 = receivers.astype(jnp.int32)
    send32 = senders.astype(jnp.int32)

    pr, ps, c = pl.pallas_call(
        _proj_body,
        out_shape=[jax.ShapeDtypeStruct((N, d_out), f32),
                   jax.ShapeDtypeStruct((N, d_out), f32),
                   jax.ShapeDtypeStruct((1, d_out), f32)],
    )(nodes, wr, ws, globals_, wg, b2)

    epw = E // _NW
    nchunk = epw // _CHUNK
    grows = E // pack
    mesh = plsc.VectorSubcoreMesh(core_axis_name="c", subcore_axis_name="s")
    sc = pl.kernel(
        functools.partial(_sc_body, epw, nchunk, pack, d_out),
        mesh=mesh,
        compiler_params=pltpu.CompilerParams(use_tc_tiling_on_sc=False),
        out_type=jax.ShapeDtypeStruct((grows, 128), f32),
        scratch_types=[
            pltpu.VMEM((epw,), jnp.int32),
            pltpu.VMEM((epw,), jnp.int32),
            pltpu.VMEM((_CHUNK, d_out), f32),
            pltpu.VMEM((_CHUNK, d_out), f32),
            pltpu.VMEM((_CHUNK, d_out), f32),
            pltpu.VMEM((_CHUNK, d_out), f32),
            pltpu.SemaphoreType.DMA,
            pltpu.SemaphoreType.DMA,
            pltpu.SemaphoreType.DMA,
            pltpu.SemaphoreType.DMA,
            pltpu.SemaphoreType.DMA,
            pltpu.SemaphoreType.DMA,
        ],
    )
    g = sc(pr, ps, recv32, send32)

    et = edges.T
    wet = we.T
    ct = c.T
    eye16 = jnp.eye(d_out, dtype=f32)
    eb = pack * _BQ
    out_t = pl.pallas_call(
        functools.partial(_combine_body, pack),
        grid=(E // eb,),
        in_specs=[
            pl.BlockSpec((d_out, eb), lambda i: (0, i)),
            pl.BlockSpec((_BQ, 128), lambda i: (i, 0)),
            pl.BlockSpec((d_out, d_out), lambda i: (0, 0)),
            pl.BlockSpec((d_out, 1), lambda i: (0, 0)),
            pl.BlockSpec((d_out, d_out), lambda i: (0, 0)),
        ],
        out_specs=pl.BlockSpec((d_out, eb), lambda i: (0, i)),
        out_shape=jax.ShapeDtypeStruct((d_out, E), f32),
    )(et, g, wet, ct, eye16)
    return out_t.T


# speedup vs baseline: 1.4726x; 1.1287x over previous
import functools

import jax
import jax.numpy as jnp
from jax import lax
from jax.experimental import pallas as pl
from jax.experimental.pallas import tpu as pltpu
from jax.experimental.pallas import tpu_sc as plsc

_NC = 2
_NS = 16
_NW = _NC * _NS
_CHUNK = 400
_BQ = 2000


def _proj_body(nodes_ref, wr_ref, ws_ref, glob_ref, wg_ref, b_ref,
               pr_ref, ps_ref, c_ref):
    n = nodes_ref[...]
    hi = jax.lax.Precision.HIGHEST
    pr_ref[...] = jnp.dot(n, wr_ref[...], precision=hi,
                          preferred_element_type=jnp.float32)
    ps_ref[...] = jnp.dot(n, ws_ref[...], precision=hi,
                          preferred_element_type=jnp.float32)
    c_ref[...] = jnp.dot(glob_ref[...], wg_ref[...], precision=hi,
                         preferred_element_type=jnp.float32) + b_ref[...]


def _sc_body(epw, nchunk, pack, d_out, pr_hbm, ps_hbm, recv_hbm, send_hbm,
             g_hbm, ridx, sidx, rrows0, srows0, rrows1, srows1,
             sem_r0, sem_s0, sem_st0, sem_r1, sem_s1, sem_st1):
    wid = lax.axis_index("s") * _NC + lax.axis_index("c")
    base = wid * epw
    bufs = ((rrows0, srows0, sem_r0, sem_s0, sem_st0),
            (rrows1, srows1, sem_r1, sem_s1, sem_st1))

    pltpu.sync_copy(recv_hbm.at[pl.ds(base, epw)], ridx)
    pltpu.sync_copy(send_hbm.at[pl.ds(base, epw)], sidx)

    def issue(ci, b):
        rr, sr, semr, sems, _ = bufs[b]
        sl = pl.ds(ci * _CHUNK, _CHUNK)
        pltpu.async_copy(pr_hbm.at[ridx.at[sl]], rr, semr)
        pltpu.async_copy(ps_hbm.at[sidx.at[sl]], sr, sems)

    def finish(ci, b):
        rr, sr, semr, sems, semst = bufs[b]
        sl = pl.ds(0, _CHUNK)
        pltpu.make_async_copy(pr_hbm.at[ridx.at[sl]], rr, semr).wait()
        pltpu.make_async_copy(ps_hbm.at[sidx.at[sl]], sr, sems).wait()

        def row_body(r, carry2):
            e = r * 4
            for k in range(4):
                rr[e + k, :] = rr[e + k, :] + sr[e + k, :]
            return carry2

        lax.fori_loop(0, _CHUNK // 4, row_body, 0)
        off = base + ci * _CHUNK
        band = (off // _BQ) % pack
        r0 = (off // (_BQ * pack)) * _BQ + (off % _BQ)
        pltpu.async_copy(
            rr, g_hbm.at[pl.ds(r0, _CHUNK), pl.ds(band * d_out, d_out)],
            semst)

    def drain_store(b):
        rr, _, _, _, semst = bufs[b]
        pltpu.make_async_copy(
            rr, g_hbm.at[pl.ds(0, _CHUNK), pl.ds(0, d_out)], semst).wait()

    issue(0, 0)
    issue(1, 1)

    def pair_body(j, carry):
        for h in range(2):
            ci = j * 2 + h

            finish(ci, h)

            @pl.when(ci + 2 < nchunk)
            def _(h=h, ci=ci):
                drain_store(h)
                issue(ci + 2, h)

        return carry

    lax.fori_loop(0, (nchunk - 1) // 2, pair_body, 0)
    finish(nchunk - 1, 0)
    drain_store(0)
    drain_store(1)


def _combine_body(pack, et_ref, g_ref, wet_ref, ct_ref, eye_ref, ot_ref):
    hi = jax.lax.Precision.HIGHEST
    edge_term = jnp.dot(wet_ref[...], et_ref[...], precision=hi,
                        preferred_element_type=jnp.float32)
    d_out = 128 // pack
    ct = ct_ref[...]
    g_t = jax.lax.dot_general(
        eye_ref[...], g_ref[...], (((1,), (1,)), ((), ())),
        preferred_element_type=jnp.float32)
    for k in range(pack):
        ot_ref[:, k * _BQ:(k + 1) * _BQ] = (
            edge_term[:, k * _BQ:(k + 1) * _BQ]
            + g_t[k * d_out:(k + 1) * d_out, :] + ct)


def kernel(edges, nodes, globals_, receivers, senders, W, b):
    E, d_edge = edges.shape
    N, d_node = nodes.shape
    d_out = W.shape[-1]
    f32 = jnp.float32
    pack = 128 // d_out

    we = W[:d_edge]
    wr = W[d_edge:d_edge + d_node]
    ws = W[d_edge + d_node:d_edge + 2 * d_node]
    wg = W[d_edge + 2 * d_node:]
    b2 = b.reshape(1, d_out)

    recv32 = receivers.astype(jnp.int32)
    send32 = senders.astype(jnp.int32)

    pr, ps, c = pl.pallas_call(
        _proj_body,
        out_shape=[jax.ShapeDtypeStruct((N, d_out), f32),
                   jax.ShapeDtypeStruct((N, d_out), f32),
                   jax.ShapeDtypeStruct((1, d_out), f32)],
    )(nodes, wr, ws, globals_, wg, b2)

    epw = E // _NW
    nchunk = epw // _CHUNK
    grows = E // pack
    mesh = plsc.VectorSubcoreMesh(core_axis_name="c", subcore_axis_name="s")
    sc = pl.kernel(
        functools.partial(_sc_body, epw, nchunk, pack, d_out),
        mesh=mesh,
        compiler_params=pltpu.CompilerParams(use_tc_tiling_on_sc=False),
        out_type=jax.ShapeDtypeStruct((grows, 128), f32),
        scratch_types=[
            pltpu.VMEM((epw,), jnp.int32),
            pltpu.VMEM((epw,), jnp.int32),
            pltpu.VMEM((_CHUNK, d_out), f32),
            pltpu.VMEM((_CHUNK, d_out), f32),
            pltpu.VMEM((_CHUNK, d_out), f32),
            pltpu.VMEM((_CHUNK, d_out), f32),
            pltpu.SemaphoreType.DMA,
            pltpu.SemaphoreType.DMA,
            pltpu.SemaphoreType.DMA,
            pltpu.SemaphoreType.DMA,
            pltpu.SemaphoreType.DMA,
            pltpu.SemaphoreType.DMA,
        ],
    )
    g = sc(pr, ps, recv32, send32)

    et = edges.T
    wet = we.T
    ct = c.T
    eye128 = jnp.eye(128, dtype=f32)
    eb = pack * _BQ
    out_t = pl.pallas_call(
        functools.partial(_combine_body, pack),
        grid=(E // eb,),
        in_specs=[
            pl.BlockSpec((d_out, eb), lambda i: (0, i)),
            pl.BlockSpec((_BQ, 128), lambda i: (i, 0)),
            pl.BlockSpec((d_out, d_out), lambda i: (0, 0)),
            pl.BlockSpec((d_out, 1), lambda i: (0, 0)),
            pl.BlockSpec((128, 128), lambda i: (0, 0)),
        ],
        out_specs=pl.BlockSpec((d_out, eb), lambda i: (0, i)),
        out_shape=jax.ShapeDtypeStruct((d_out, E), f32),
    )(et, g, wet, ct, eye128)
    return out_t.T


# speedup vs baseline: 1.5300x; 1.0390x over previous
import functools

import jax
import jax.numpy as jnp
from jax import lax
from jax.experimental import pallas as pl
from jax.experimental.pallas import tpu as pltpu
from jax.experimental.pallas import tpu_sc as plsc

_NC = 2
_NS = 16
_NW = _NC * _NS
_CHUNK = 400
_BQ = 4000


def _proj_body(nodes_ref, wr_ref, ws_ref, glob_ref, wg_ref, b_ref,
               pr_ref, ps_ref, c_ref):
    n = nodes_ref[...]
    hi = jax.lax.Precision.HIGHEST
    pr_ref[...] = jnp.dot(n, wr_ref[...], precision=hi,
                          preferred_element_type=jnp.float32)
    ps_ref[...] = jnp.dot(n, ws_ref[...], precision=hi,
                          preferred_element_type=jnp.float32)
    c_ref[...] = jnp.dot(glob_ref[...], wg_ref[...], precision=hi,
                         preferred_element_type=jnp.float32) + b_ref[...]


def _sc_body(epw, nchunk, pack, d_out, pr_hbm, ps_hbm, recv_hbm, send_hbm,
             g_hbm, ridx, sidx, rrows0, srows0, rrows1, srows1,
             sem_r0, sem_s0, sem_st0, sem_r1, sem_s1, sem_st1):
    wid = lax.axis_index("s") * _NC + lax.axis_index("c")
    base = wid * epw
    bufs = ((rrows0, srows0, sem_r0, sem_s0, sem_st0),
            (rrows1, srows1, sem_r1, sem_s1, sem_st1))

    pltpu.sync_copy(recv_hbm.at[pl.ds(base, epw)], ridx)
    pltpu.sync_copy(send_hbm.at[pl.ds(base, epw)], sidx)

    def issue(ci, b):
        rr, sr, semr, sems, _ = bufs[b]
        sl = pl.ds(ci * _CHUNK, _CHUNK)
        pltpu.async_copy(pr_hbm.at[ridx.at[sl]], rr, semr)
        pltpu.async_copy(ps_hbm.at[sidx.at[sl]], sr, sems)

    def finish(ci, b):
        rr, sr, semr, sems, semst = bufs[b]
        sl = pl.ds(0, _CHUNK)
        pltpu.make_async_copy(pr_hbm.at[ridx.at[sl]], rr, semr).wait()
        pltpu.make_async_copy(ps_hbm.at[sidx.at[sl]], sr, sems).wait()

        def row_body(r, carry2):
            e = r * 4
            for k in range(4):
                rr[e + k, :] = rr[e + k, :] + sr[e + k, :]
            return carry2

        lax.fori_loop(0, _CHUNK // 4, row_body, 0)
        off = base + ci * _CHUNK
        band = (off // _BQ) % pack
        r0 = (off // (_BQ * pack)) * _BQ + (off % _BQ)
        pltpu.async_copy(
            rr, g_hbm.at[pl.ds(r0, _CHUNK), pl.ds(band * d_out, d_out)],
            semst)

    def drain_store(b):
        rr, _, _, _, semst = bufs[b]
        pltpu.make_async_copy(
            rr, g_hbm.at[pl.ds(0, _CHUNK), pl.ds(0, d_out)], semst).wait()

    issue(0, 0)
    issue(1, 1)

    def pair_body(j, carry):
        for h in range(2):
            ci = j * 2 + h

            finish(ci, h)

            @pl.when(ci + 2 < nchunk)
            def _(h=h, ci=ci):
                drain_store(h)
                issue(ci + 2, h)

        return carry

    lax.fori_loop(0, (nchunk - 1) // 2, pair_body, 0)
    finish(nchunk - 1, 0)
    drain_store(0)
    drain_store(1)


def _combine_body(pack, et_ref, g_ref, wet_ref, ct_ref, eye_ref, ot_ref):
    hi = jax.lax.Precision.HIGHEST
    edge_term = jnp.dot(wet_ref[...], et_ref[...], precision=hi,
                        preferred_element_type=jnp.float32)
    d_out = 128 // pack
    ct = ct_ref[...]
    g_t = jax.lax.dot_general(
        eye_ref[...], g_ref[...], (((1,), (1,)), ((), ())),
        preferred_element_type=jnp.float32)
    for k in range(pack):
        ot_ref[:, k * _BQ:(k + 1) * _BQ] = (
            edge_term[:, k * _BQ:(k + 1) * _BQ]
            + g_t[k * d_out:(k + 1) * d_out, :] + ct)


def kernel(edges, nodes, globals_, receivers, senders, W, b):
    E, d_edge = edges.shape
    N, d_node = nodes.shape
    d_out = W.shape[-1]
    f32 = jnp.float32
    pack = 128 // d_out

    we = W[:d_edge]
    wr = W[d_edge:d_edge + d_node]
    ws = W[d_edge + d_node:d_edge + 2 * d_node]
    wg = W[d_edge + 2 * d_node:]
    b2 = b.reshape(1, d_out)

    recv32 = receivers.astype(jnp.int32)
    send32 = senders.astype(jnp.int32)

    pr, ps, c = pl.pallas_call(
        _proj_body,
        out_shape=[jax.ShapeDtypeStruct((N, d_out), f32),
                   jax.ShapeDtypeStruct((N, d_out), f32),
                   jax.ShapeDtypeStruct((1, d_out), f32)],
    )(nodes, wr, ws, globals_, wg, b2)

    epw = E // _NW
    nchunk = epw // _CHUNK
    grows = E // pack
    mesh = plsc.VectorSubcoreMesh(core_axis_name="c", subcore_axis_name="s")
    sc = pl.kernel(
        functools.partial(_sc_body, epw, nchunk, pack, d_out),
        mesh=mesh,
        compiler_params=pltpu.CompilerParams(use_tc_tiling_on_sc=False),
        out_type=jax.ShapeDtypeStruct((grows, 128), f32),
        scratch_types=[
            pltpu.VMEM((epw,), jnp.int32),
            pltpu.VMEM((epw,), jnp.int32),
            pltpu.VMEM((_CHUNK, d_out), f32),
            pltpu.VMEM((_CHUNK, d_out), f32),
            pltpu.VMEM((_CHUNK, d_out), f32),
            pltpu.VMEM((_CHUNK, d_out), f32),
            pltpu.SemaphoreType.DMA,
            pltpu.SemaphoreType.DMA,
            pltpu.SemaphoreType.DMA,
            pltpu.SemaphoreType.DMA,
            pltpu.SemaphoreType.DMA,
            pltpu.SemaphoreType.DMA,
        ],
    )
    g = sc(pr, ps, recv32, send32)

    et = edges.T
    wet = we.T
    ct = c.T
    eye128 = jnp.eye(128, dtype=f32)
    eb = pack * _BQ
    out_t = pl.pallas_call(
        functools.partial(_combine_body, pack),
        grid=(E // eb,),
        in_specs=[
            pl.BlockSpec((d_out, eb), lambda i: (0, i)),
            pl.BlockSpec((_BQ, 128), lambda i: (i, 0)),
            pl.BlockSpec((d_out, d_out), lambda i: (0, 0)),
            pl.BlockSpec((d_out, 1), lambda i: (0, 0)),
            pl.BlockSpec((128, 128), lambda i: (0, 0)),
        ],
        out_specs=pl.BlockSpec((d_out, eb), lambda i: (0, i)),
        out_shape=jax.ShapeDtypeStruct((d_out, E), f32),
    )(et, g, wet, ct, eye128)
    return out_t.T
